# baseline (speedup 1.0000x reference)
import functools

import numpy as np
import jax
import jax.numpy as jnp
from jax import lax
from jax.experimental import pallas as pl
from jax.experimental.pallas import tpu as pltpu
from jax.experimental.pallas import tpu_sc as plsc

N_J = 80000
N_T = 20000
_PAD = 2048
_PADT = 2528
NJP = N_J + _PAD
NTP = N_T + _PADT
NC = 2
NS = 16
NW = NC * NS
CH = 10240
TJ_CH = 2
JJ_CH = 6
E_TJ_P = NW * TJ_CH * CH
E_JJ_P = NW * JJ_CH * CH
RPT = NJP // NS
TPT = NTP // NS
_SP_BIAS = float(np.log(np.exp(1.0) - 1.0))

T_YJ0 = 0
T_YJ1 = NJP
T_YT0 = 2 * NJP
T_YT1 = 2 * NJP + NTP
T_I0 = 2 * NJP + 2 * NTP
T_I1 = T_I0 + 2 * NJP
E_STJ = 0
E_DTJ = E_TJ_P
E_SJJ = 2 * E_TJ_P
E_DJJ = 2 * E_TJ_P + E_JJ_P


def _pre_joint_body(woutT_ref, wreljjT_ref, wrtjT_ref, wrjjT_ref, wjT_ref,
                    bj_ref, brel_ref, bout_ref, x_ref, o_ref):
    ajjT = jnp.dot(woutT_ref[...], wreljjT_ref[...],
                   preferred_element_type=jnp.float32)
    arT = jnp.dot(woutT_ref[...], wrtjT_ref[...] + wrjjT_ref[...],
                  preferred_element_type=jnp.float32)
    gjT = jnp.dot(ajjT, wjT_ref[...], preferred_element_type=jnp.float32)
    grT = jnp.dot(arT, wjT_ref[...], preferred_element_type=jnp.float32)
    cj = jnp.dot(ajjT, bj_ref[...], preferred_element_type=jnp.float32)
    cr = (jnp.dot(arT, bj_ref[...], preferred_element_type=jnp.float32)
          + jnp.dot(woutT_ref[...], brel_ref[...],
                    preferred_element_type=jnp.float32)
          + bout_ref[...])
    g4 = jnp.concatenate([gjT, grT], axis=0)
    c4 = jnp.concatenate([cj, cr], axis=0)
    x = x_ref[...]
    o_ref[...] = lax.dot_general(g4, x, (((1,), (1,)), ((), ())),
                                 preferred_element_type=jnp.float32) + c4


def _pre_torso_body(woutT_ref, wreltjT_ref, wtT_ref, bt_ref, x_ref, o_ref):
    atjT = jnp.dot(woutT_ref[...], wreltjT_ref[...],
                   preferred_element_type=jnp.float32)
    gtT = jnp.dot(atjT, wtT_ref[...], preferred_element_type=jnp.float32)
    ct = jnp.dot(atjT, bt_ref[...], preferred_element_type=jnp.float32)
    x = x_ref[...]
    o_ref[...] = lax.dot_general(gtT, x, (((1,), (1,)), ((), ())),
                                 preferred_element_type=jnp.float32) + ct


def _post_body(p0_ref, p1_ref, loc_ref, scale_ref):
    loc_ref[...] = p0_ref[0:1, :] + p0_ref[1:2, :]
    s = p1_ref[0:1, :] + p1_ref[1:2, :] + _SP_BIAS
    scale_ref[...] = jax.nn.softplus(s)


def _sc_body(tabs, edges, out0, out1,
             src_a, src_b, dst_a, dst_b, g0_a, g0_b, g1_a, g1_b,
             stage_v, acc0, acc1, ts0, ts1, js0, js1, isem, gsem, ssem):
    src_v = (src_a, src_b)
    dst_v = (dst_a, dst_b)
    g0_v = (g0_a, g0_b)
    g1_v = (g1_a, g1_b)
    c = lax.axis_index("c")
    s = lax.axis_index("s")
    wid = s * NC + c
    base = s * RPT
    tb = s * TPT
    hb = c * NJP + base

    jobs = ((T_I0 + hb, acc0, base, RPT),
            (T_I1 + hb, acc1, base, RPT),
            (T_YJ0 + base, js0, base, RPT),
            (T_YJ1 + base, js1, base, RPT),
            (T_YT0 + tb, ts0, tb, TPT),
            (T_YT1 + tb, ts1, tb, TPT))
    bufs = (g0_a, g0_b, g1_a, g1_b, stage_v)
    st = {}
    ldh = {0: pltpu.async_copy(tabs.at[pl.ds(jobs[0][0], jobs[0][3])],
                               bufs[0].at[pl.ds(0, jobs[0][3])], isem)}
    for k in range(len(jobs)):
        so, dref, doff, n = jobs[k]
        if k + 1 < len(jobs):
            so2, _, _, n2 = jobs[k + 1]
            if (k + 1) - len(bufs) in st:
                st.pop((k + 1) - len(bufs)).wait()
            ldh[k + 1] = pltpu.async_copy(
                tabs.at[pl.ds(so2, n2)],
                bufs[(k + 1) % len(bufs)].at[pl.ds(0, n2)], isem)
        ldh.pop(k).wait()
        st[k] = pltpu.async_copy(bufs[k % len(bufs)].at[pl.ds(0, n)],
                                 dref.at[pl.ds(doff, n)], gsem)
    for h in st.values():
        h.wait()
    plsc.subcore_barrier()

    def do_edges(soff, doff, t0, t1, nchunks):
        def start_idx(i, b):
            off = (wid * nchunks + i) * CH
            return (
                pltpu.async_copy(edges.at[pl.ds(soff + off, CH)], src_v[b], isem),
                pltpu.async_copy(edges.at[pl.ds(doff + off, CH)], dst_v[b], isem),
            )

        ih = {0: start_idx(0, 0)}
        sh = {}
        for i in range(nchunks):
            b = i % 2
            for h in ih.pop(i):
                h.wait()
            gh = (pltpu.async_copy(t0.at[src_v[b]], g0_v[b], gsem),
                  pltpu.async_copy(t1.at[src_v[b]], g1_v[b], gsem))
            if i - 1 in sh:
                for h in sh.pop(i - 1):
                    h.wait()
            if i + 1 < nchunks:
                ih[i + 1] = start_idx(i + 1, 1 - b)
            for h in gh:
                h.wait()
            sh[i] = (pltpu.async_copy(g0_v[b], acc0.at[dst_v[b]],
                                      ssem, add=True),
                     pltpu.async_copy(g1_v[b], acc1.at[dst_v[b]],
                                      ssem, add=True))
        for hs in sh.values():
            for h in hs:
                h.wait()

    do_edges(E_STJ, E_DTJ, ts0, ts1, TJ_CH)
    do_edges(E_SJJ, E_DJJ, js0, js1, JJ_CH)
    plsc.subcore_barrier()
    h0 = pltpu.async_copy(acc0.at[pl.ds(base, RPT)],
                          stage_v.at[pl.ds(0, RPT)], gsem)
    h1 = pltpu.async_copy(acc1.at[pl.ds(base, RPT)],
                          g0_a.at[pl.ds(0, RPT)], gsem)
    h0.wait()
    s0 = pltpu.async_copy(stage_v.at[pl.ds(0, RPT)],
                          out0.at[pl.ds(hb, RPT)], ssem)
    h1.wait()
    s1 = pltpu.async_copy(g0_a.at[pl.ds(0, RPT)],
                          out1.at[pl.ds(hb, RPT)], ssem)
    s0.wait()
    s1.wait()


_sc_scatter = functools.partial(
    pl.kernel,
    mesh=plsc.VectorSubcoreMesh(core_axis_name="c", subcore_axis_name="s"),
    out_type=[jax.ShapeDtypeStruct((NC * NJP,), jnp.float32),
              jax.ShapeDtypeStruct((NC * NJP,), jnp.float32)],
    scratch_types=[
        pltpu.VMEM((CH,), jnp.int32),
        pltpu.VMEM((CH,), jnp.int32),
        pltpu.VMEM((CH,), jnp.int32),
        pltpu.VMEM((CH,), jnp.int32),
        pltpu.VMEM((CH,), jnp.float32),
        pltpu.VMEM((CH,), jnp.float32),
        pltpu.VMEM((CH,), jnp.float32),
        pltpu.VMEM((CH,), jnp.float32),
        pltpu.VMEM((RPT,), jnp.float32),
        pltpu.VMEM_SHARED((NJP,), jnp.float32),
        pltpu.VMEM_SHARED((NJP,), jnp.float32),
        pltpu.VMEM_SHARED((NTP,), jnp.float32),
        pltpu.VMEM_SHARED((NTP,), jnp.float32),
        pltpu.VMEM_SHARED((NJP,), jnp.float32),
        pltpu.VMEM_SHARED((NJP,), jnp.float32),
        pltpu.SemaphoreType.DMA,
        pltpu.SemaphoreType.DMA,
        pltpu.SemaphoreType.DMA,
    ],
)(_sc_body)


def kernel(x_joint, x_torso, edge_index_tj, edge_index_jt, edge_index_jj,
           Wj, bj, Wt, bt, Wrel_tj, brel_tj, Wroot_tj,
           Wrel_jt, brel_jt, Wroot_jt, Wrel_jj, brel_jj, Wroot_jj,
           Wout, bout):
    f32 = jnp.float32
    woutT = Wout.T
    brel_col = (brel_tj + brel_jj).reshape(64, 1)
    bj_col = bj.reshape(11, 1)
    bt_col = bt.reshape(11, 1)
    bout_col = bout.reshape(2, 1)

    blkj = 16000
    prej = pl.pallas_call(
        _pre_joint_body,
        grid=(N_J // blkj,),
        in_specs=[pl.BlockSpec((2, 64), lambda i: (0, 0)),
                  pl.BlockSpec((64, 11), lambda i: (0, 0)),
                  pl.BlockSpec((64, 11), lambda i: (0, 0)),
                  pl.BlockSpec((64, 11), lambda i: (0, 0)),
                  pl.BlockSpec((11, 2), lambda i: (0, 0)),
                  pl.BlockSpec((11, 1), lambda i: (0, 0)),
                  pl.BlockSpec((64, 1), lambda i: (0, 0)),
                  pl.BlockSpec((2, 1), lambda i: (0, 0)),
                  pl.BlockSpec((blkj, 2), lambda i: (i, 0))],
        out_specs=pl.BlockSpec((4, blkj), lambda i: (0, i)),
        out_shape=jax.ShapeDtypeStruct((4, N_J), f32),
    )(woutT, Wrel_jj.T, Wroot_tj.T, Wroot_jj.T, Wj.T,
      bj_col, brel_col, bout_col, x_joint)

    pret = pl.pallas_call(
        _pre_torso_body,
        grid=(1,),
        in_specs=[pl.BlockSpec((2, 64), lambda i: (0, 0)),
                  pl.BlockSpec((64, 11), lambda i: (0, 0)),
                  pl.BlockSpec((11, 11), lambda i: (0, 0)),
                  pl.BlockSpec((11, 1), lambda i: (0, 0)),
                  pl.BlockSpec((N_T, 11), lambda i: (0, 0))],
        out_specs=pl.BlockSpec((2, N_T), lambda i: (0, 0)),
        out_shape=jax.ShapeDtypeStruct((2, N_T), f32),
    )(woutT, Wrel_tj.T, Wt.T, bt_col, x_torso)

    zpadj = jnp.zeros((_PAD,), f32)
    zpadt = jnp.zeros((_PADT,), f32)
    zrow = jnp.zeros((NJP,), f32)
    tabs = jnp.concatenate([
        prej[0], zpadj,
        prej[1], zpadj,
        pret[0], zpadt,
        pret[1], zpadt,
        prej[2], zpadj, zrow,
        prej[3], zpadj, zrow,
    ])
    ar_tj = jnp.arange(E_TJ_P - edge_index_tj.shape[1], dtype=jnp.int32)
    ar_jj = jnp.arange(E_JJ_P - edge_index_jj.shape[1], dtype=jnp.int32)
    edges = jnp.concatenate([
        edge_index_tj[0].astype(jnp.int32), N_T + ar_tj % _PADT,
        edge_index_tj[1].astype(jnp.int32), N_J + ar_tj % _PAD,
        edge_index_jj[0].astype(jnp.int32), N_J + ar_jj % _PAD,
        edge_index_jj[1].astype(jnp.int32), N_J + ar_jj % _PAD,
    ])

    p0, p1 = _sc_scatter(tabs, edges)
    p0 = p0.reshape(NC, NJP)
    p1 = p1.reshape(NC, NJP)

    blk = 16000
    loc2, scale2 = pl.pallas_call(
        _post_body,
        grid=(N_J // blk,),
        in_specs=[pl.BlockSpec((2, blk), lambda i: (0, i)),
                  pl.BlockSpec((2, blk), lambda i: (0, i))],
        out_specs=[pl.BlockSpec((1, blk), lambda i: (0, i)),
                   pl.BlockSpec((1, blk), lambda i: (0, i))],
        out_shape=[jax.ShapeDtypeStruct((1, N_J), f32),
                   jax.ShapeDtypeStruct((1, N_J), f32)],
    )(p0, p1)
    return (loc2.reshape(N_J), scale2.reshape(N_J))


# speedup vs baseline: 1.1337x; 1.1337x over previous
import functools

import numpy as np
import jax
import jax.numpy as jnp
from jax import lax
from jax.experimental import pallas as pl
from jax.experimental.pallas import tpu as pltpu
from jax.experimental.pallas import tpu_sc as plsc

N_J = 80000
N_T = 20000
_PAD = 2048
_PADT = 2528
NJP = N_J + _PAD
NTP = N_T + _PADT
NC = 2
NS = 16
NW = NC * NS
CH = 10240
TJ_CH = 2
JJ_CH = 6
E_TJ_P = NW * TJ_CH * CH
E_JJ_P = NW * JJ_CH * CH
RPT = NJP // NS
TPT = NTP // NS
_SP_BIAS = float(np.log(np.exp(1.0) - 1.0))

T_YJ0 = 0
T_YJ1 = NJP
T_YT0 = 2 * NJP
T_YT1 = 2 * NJP + NTP
T_I0 = 2 * NJP + 2 * NTP
T_I1 = T_I0 + 2 * NJP
E_STJ = 0
E_DTJ = E_TJ_P
E_SJJ = 2 * E_TJ_P
E_DJJ = 2 * E_TJ_P + E_JJ_P


def _pre_joint_body(woutT_ref, wreljjT_ref, wrtjT_ref, wrjjT_ref, wjT_ref,
                    bj_ref, brel_ref, bout_ref, x_ref, o_ref):
    ajjT = jnp.dot(woutT_ref[...], wreljjT_ref[...],
                   preferred_element_type=jnp.float32)
    arT = jnp.dot(woutT_ref[...], wrtjT_ref[...] + wrjjT_ref[...],
                  preferred_element_type=jnp.float32)
    gjT = jnp.dot(ajjT, wjT_ref[...], preferred_element_type=jnp.float32)
    grT = jnp.dot(arT, wjT_ref[...], preferred_element_type=jnp.float32)
    cj = jnp.dot(ajjT, bj_ref[...], preferred_element_type=jnp.float32)
    cr = (jnp.dot(arT, bj_ref[...], preferred_element_type=jnp.float32)
          + jnp.dot(woutT_ref[...], brel_ref[...],
                    preferred_element_type=jnp.float32)
          + bout_ref[...])
    g4 = jnp.concatenate([gjT, grT], axis=0)
    c4 = jnp.concatenate([cj, cr], axis=0)
    x = x_ref[...]
    o_ref[...] = jnp.dot(g4, x, preferred_element_type=jnp.float32) + c4


def _pre_torso_body(woutT_ref, wreltjT_ref, wtT_ref, bt_ref, x_ref, o_ref):
    atjT = jnp.dot(woutT_ref[...], wreltjT_ref[...],
                   preferred_element_type=jnp.float32)
    gtT = jnp.dot(atjT, wtT_ref[...], preferred_element_type=jnp.float32)
    ct = jnp.dot(atjT, bt_ref[...], preferred_element_type=jnp.float32)
    x = x_ref[...]
    o_ref[...] = jnp.dot(gtT, x, preferred_element_type=jnp.float32) + ct


def _post_body(p0_ref, p1_ref, loc_ref, scale_ref):
    loc_ref[...] = p0_ref[0:1, :] + p0_ref[1:2, :]
    s = p1_ref[0:1, :] + p1_ref[1:2, :] + _SP_BIAS
    scale_ref[...] = jax.nn.softplus(s)


def _sc_body(tabs, edges, out0, out1,
             src_a, src_b, dst_a, dst_b, g0_a, g0_b, g1_a, g1_b,
             stage_v, acc0, acc1, ts0, ts1, js0, js1, isem, gsem, ssem):
    src_v = (src_a, src_b)
    dst_v = (dst_a, dst_b)
    g0_v = (g0_a, g0_b)
    g1_v = (g1_a, g1_b)
    c = lax.axis_index("c")
    s = lax.axis_index("s")
    wid = s * NC + c
    base = s * RPT
    tb = s * TPT
    hb = c * NJP + base

    jobs = ((T_I0 + hb, acc0, base, RPT),
            (T_I1 + hb, acc1, base, RPT),
            (T_YJ0 + base, js0, base, RPT),
            (T_YJ1 + base, js1, base, RPT),
            (T_YT0 + tb, ts0, tb, TPT),
            (T_YT1 + tb, ts1, tb, TPT))
    bufs = (g0_a, g0_b, g1_a, g1_b, stage_v)
    st = {}
    ldh = {0: pltpu.async_copy(tabs.at[pl.ds(jobs[0][0], jobs[0][3])],
                               bufs[0].at[pl.ds(0, jobs[0][3])], isem)}
    for k in range(len(jobs)):
        so, dref, doff, n = jobs[k]
        if k + 1 < len(jobs):
            so2, _, _, n2 = jobs[k + 1]
            if (k + 1) - len(bufs) in st:
                st.pop((k + 1) - len(bufs)).wait()
            ldh[k + 1] = pltpu.async_copy(
                tabs.at[pl.ds(so2, n2)],
                bufs[(k + 1) % len(bufs)].at[pl.ds(0, n2)], isem)
        ldh.pop(k).wait()
        st[k] = pltpu.async_copy(bufs[k % len(bufs)].at[pl.ds(0, n)],
                                 dref.at[pl.ds(doff, n)], gsem)
    for h in st.values():
        h.wait()
    plsc.subcore_barrier()

    def do_edges(soff, doff, t0, t1, nchunks):
        def start_idx(i, b):
            off = (wid * nchunks + i) * CH
            return (
                pltpu.async_copy(edges.at[pl.ds(soff + off, CH)], src_v[b], isem),
                pltpu.async_copy(edges.at[pl.ds(doff + off, CH)], dst_v[b], isem),
            )

        ih = {0: start_idx(0, 0)}
        sh = {}
        for i in range(nchunks):
            b = i % 2
            for h in ih.pop(i):
                h.wait()
            gh = (pltpu.async_copy(t0.at[src_v[b]], g0_v[b], gsem),
                  pltpu.async_copy(t1.at[src_v[b]], g1_v[b], gsem))
            if i - 1 in sh:
                for h in sh.pop(i - 1):
                    h.wait()
            if i + 1 < nchunks:
                ih[i + 1] = start_idx(i + 1, 1 - b)
            for h in gh:
                h.wait()
            sh[i] = (pltpu.async_copy(g0_v[b], acc0.at[dst_v[b]],
                                      ssem, add=True),
                     pltpu.async_copy(g1_v[b], acc1.at[dst_v[b]],
                                      ssem, add=True))
        for hs in sh.values():
            for h in hs:
                h.wait()

    do_edges(E_STJ, E_DTJ, ts0, ts1, TJ_CH)
    do_edges(E_SJJ, E_DJJ, js0, js1, JJ_CH)
    plsc.subcore_barrier()
    h0 = pltpu.async_copy(acc0.at[pl.ds(base, RPT)],
                          stage_v.at[pl.ds(0, RPT)], gsem)
    h1 = pltpu.async_copy(acc1.at[pl.ds(base, RPT)],
                          g0_a.at[pl.ds(0, RPT)], gsem)
    h0.wait()
    s0 = pltpu.async_copy(stage_v.at[pl.ds(0, RPT)],
                          out0.at[pl.ds(hb, RPT)], ssem)
    h1.wait()
    s1 = pltpu.async_copy(g0_a.at[pl.ds(0, RPT)],
                          out1.at[pl.ds(hb, RPT)], ssem)
    s0.wait()
    s1.wait()


_sc_scatter = functools.partial(
    pl.kernel,
    mesh=plsc.VectorSubcoreMesh(core_axis_name="c", subcore_axis_name="s"),
    out_type=[jax.ShapeDtypeStruct((NC * NJP,), jnp.float32),
              jax.ShapeDtypeStruct((NC * NJP,), jnp.float32)],
    scratch_types=[
        pltpu.VMEM((CH,), jnp.int32),
        pltpu.VMEM((CH,), jnp.int32),
        pltpu.VMEM((CH,), jnp.int32),
        pltpu.VMEM((CH,), jnp.int32),
        pltpu.VMEM((CH,), jnp.float32),
        pltpu.VMEM((CH,), jnp.float32),
        pltpu.VMEM((CH,), jnp.float32),
        pltpu.VMEM((CH,), jnp.float32),
        pltpu.VMEM((RPT,), jnp.float32),
        pltpu.VMEM_SHARED((NJP,), jnp.float32),
        pltpu.VMEM_SHARED((NJP,), jnp.float32),
        pltpu.VMEM_SHARED((NTP,), jnp.float32),
        pltpu.VMEM_SHARED((NTP,), jnp.float32),
        pltpu.VMEM_SHARED((NJP,), jnp.float32),
        pltpu.VMEM_SHARED((NJP,), jnp.float32),
        pltpu.SemaphoreType.DMA,
        pltpu.SemaphoreType.DMA,
        pltpu.SemaphoreType.DMA,
    ],
)(_sc_body)


def kernel(x_joint, x_torso, edge_index_tj, edge_index_jt, edge_index_jj,
           Wj, bj, Wt, bt, Wrel_tj, brel_tj, Wroot_tj,
           Wrel_jt, brel_jt, Wroot_jt, Wrel_jj, brel_jj, Wroot_jj,
           Wout, bout):
    f32 = jnp.float32
    xjT = x_joint.T
    xtT = x_torso.T
    woutT = Wout.T
    brel_col = (brel_tj + brel_jj).reshape(64, 1)
    bj_col = bj.reshape(11, 1)
    bt_col = bt.reshape(11, 1)
    bout_col = bout.reshape(2, 1)

    blkj = 16000
    prej = pl.pallas_call(
        _pre_joint_body,
        grid=(N_J // blkj,),
        in_specs=[pl.BlockSpec((2, 64), lambda i: (0, 0)),
                  pl.BlockSpec((64, 11), lambda i: (0, 0)),
                  pl.BlockSpec((64, 11), lambda i: (0, 0)),
                  pl.BlockSpec((64, 11), lambda i: (0, 0)),
                  pl.BlockSpec((11, 2), lambda i: (0, 0)),
                  pl.BlockSpec((11, 1), lambda i: (0, 0)),
                  pl.BlockSpec((64, 1), lambda i: (0, 0)),
                  pl.BlockSpec((2, 1), lambda i: (0, 0)),
                  pl.BlockSpec((2, blkj), lambda i: (0, i))],
        out_specs=pl.BlockSpec((4, blkj), lambda i: (0, i)),
        out_shape=jax.ShapeDtypeStruct((4, N_J), f32),
    )(woutT, Wrel_jj.T, Wroot_tj.T, Wroot_jj.T, Wj.T,
      bj_col, brel_col, bout_col, xjT)

    pret = pl.pallas_call(
        _pre_torso_body,
        grid=(1,),
        in_specs=[pl.BlockSpec((2, 64), lambda i: (0, 0)),
                  pl.BlockSpec((64, 11), lambda i: (0, 0)),
                  pl.BlockSpec((11, 11), lambda i: (0, 0)),
                  pl.BlockSpec((11, 1), lambda i: (0, 0)),
                  pl.BlockSpec((11, N_T), lambda i: (0, 0))],
        out_specs=pl.BlockSpec((2, N_T), lambda i: (0, 0)),
        out_shape=jax.ShapeDtypeStruct((2, N_T), f32),
    )(woutT, Wrel_tj.T, Wt.T, bt_col, xtT)

    zpadj = jnp.zeros((_PAD,), f32)
    zpadt = jnp.zeros((_PADT,), f32)
    zrow = jnp.zeros((NJP,), f32)
    tabs = jnp.concatenate([
        prej[0], zpadj,
        prej[1], zpadj,
        pret[0], zpadt,
        pret[1], zpadt,
        prej[2], zpadj, zrow,
        prej[3], zpadj, zrow,
    ])
    ar_tj = jnp.arange(E_TJ_P - edge_index_tj.shape[1], dtype=jnp.int32)
    ar_jj = jnp.arange(E_JJ_P - edge_index_jj.shape[1], dtype=jnp.int32)
    edges = jnp.concatenate([
        edge_index_tj[0].astype(jnp.int32), N_T + ar_tj % _PADT,
        edge_index_tj[1].astype(jnp.int32), N_J + ar_tj % _PAD,
        edge_index_jj[0].astype(jnp.int32), N_J + ar_jj % _PAD,
        edge_index_jj[1].astype(jnp.int32), N_J + ar_jj % _PAD,
    ])

    p0, p1 = _sc_scatter(tabs, edges)
    p0 = p0.reshape(NC, NJP)
    p1 = p1.reshape(NC, NJP)

    blk = 16000
    loc2, scale2 = pl.pallas_call(
        _post_body,
        grid=(N_J // blk,),
        in_specs=[pl.BlockSpec((2, blk), lambda i: (0, i)),
                  pl.BlockSpec((2, blk), lambda i: (0, i))],
        out_specs=[pl.BlockSpec((1, blk), lambda i: (0, i)),
                   pl.BlockSpec((1, blk), lambda i: (0, i))],
        out_shape=[jax.ShapeDtypeStruct((1, N_J), f32),
                   jax.ShapeDtypeStruct((1, N_J), f32)],
    )(p0, p1)
    return (loc2.reshape(N_J), scale2.reshape(N_J))


# speedup vs baseline: 1.2566x; 1.1084x over previous
import functools

import numpy as np
import jax
import jax.numpy as jnp
from jax import lax
from jax.experimental import pallas as pl
from jax.experimental.pallas import tpu as pltpu
from jax.experimental.pallas import tpu_sc as plsc

N_J = 80000
N_T = 20000
_PAD = 2048
_PADT = 2528
NJP = N_J + _PAD
NTP = N_T + _PADT
NC = 2
NS = 16
NW = NC * NS
CH = 10240
TJ_CH = 2
JJ_CH = 6
E_TJ_P = NW * TJ_CH * CH
E_JJ_P = NW * JJ_CH * CH
RPT = NJP // NS
TPT = NTP // NS
_SP_BIAS = float(np.log(np.exp(1.0) - 1.0))

def _pre_joint_body(woutT_ref, wreljjT_ref, wrtjT_ref, wrjjT_ref, wjT_ref,
                    bj_ref, brel_ref, bout_ref, x_ref, o_ref):
    ajjT = jnp.dot(woutT_ref[...], wreljjT_ref[...],
                   preferred_element_type=jnp.float32)
    arT = jnp.dot(woutT_ref[...], wrtjT_ref[...] + wrjjT_ref[...],
                  preferred_element_type=jnp.float32)
    gjT = jnp.dot(ajjT, wjT_ref[...], preferred_element_type=jnp.float32)
    grT = jnp.dot(arT, wjT_ref[...], preferred_element_type=jnp.float32)
    cj = jnp.dot(ajjT, bj_ref[...], preferred_element_type=jnp.float32)
    cr = (jnp.dot(arT, bj_ref[...], preferred_element_type=jnp.float32)
          + jnp.dot(woutT_ref[...], brel_ref[...],
                    preferred_element_type=jnp.float32)
          + bout_ref[...])
    g4 = jnp.concatenate([gjT, grT], axis=0)
    c4 = jnp.concatenate([cj, cr], axis=0)
    x = x_ref[...]
    o_ref[...] = jnp.dot(g4, x, preferred_element_type=jnp.float32) + c4


def _pre_torso_body(woutT_ref, wreltjT_ref, wtT_ref, bt_ref, x_ref, o_ref):
    atjT = jnp.dot(woutT_ref[...], wreltjT_ref[...],
                   preferred_element_type=jnp.float32)
    gtT = jnp.dot(atjT, wtT_ref[...], preferred_element_type=jnp.float32)
    ct = jnp.dot(atjT, bt_ref[...], preferred_element_type=jnp.float32)
    x = x_ref[...]
    o_ref[...] = jnp.dot(gtT, x, preferred_element_type=jnp.float32) + ct


def _post_body(p0_ref, p1_ref, loc_ref, scale_ref):
    loc_ref[...] = p0_ref[0:1, :] + p0_ref[1:2, :]
    s = p1_ref[0:1, :] + p1_ref[1:2, :] + _SP_BIAS
    scale_ref[...] = jax.nn.softplus(s)


def _sc_body(yt0, yt1, yj0, yj1, stj, dtj, sjj, djj, init0, init1,
             out0, out1,
             src_a, src_b, dst_a, dst_b, g0_a, g0_b, g1_a, g1_b,
             stage_v, acc0, acc1, ts0, ts1, js0, js1, isem, gsem, ssem):
    src_v = (src_a, src_b)
    dst_v = (dst_a, dst_b)
    g0_v = (g0_a, g0_b)
    g1_v = (g1_a, g1_b)
    c = lax.axis_index("c")
    s = lax.axis_index("s")
    wid = s * NC + c
    base = s * RPT
    tb = s * TPT
    hb = c * NJP + base

    jobs = ((init0, hb, acc0, base, RPT),
            (init1, hb, acc1, base, RPT),
            (yj0, base, js0, base, RPT),
            (yj1, base, js1, base, RPT),
            (yt0, tb, ts0, tb, TPT),
            (yt1, tb, ts1, tb, TPT))
    bufs = (g0_a, g0_b, g1_a, g1_b, stage_v)
    st = {}
    ldh = {0: pltpu.async_copy(jobs[0][0].at[pl.ds(jobs[0][1], jobs[0][4])],
                               bufs[0].at[pl.ds(0, jobs[0][4])], isem)}
    for k in range(len(jobs)):
        _, _, dref, doff, n = jobs[k]
        if k + 1 < len(jobs):
            sref2, so2, _, _, n2 = jobs[k + 1]
            if (k + 1) - len(bufs) in st:
                st.pop((k + 1) - len(bufs)).wait()
            ldh[k + 1] = pltpu.async_copy(
                sref2.at[pl.ds(so2, n2)],
                bufs[(k + 1) % len(bufs)].at[pl.ds(0, n2)], isem)
        ldh.pop(k).wait()
        st[k] = pltpu.async_copy(bufs[k % len(bufs)].at[pl.ds(0, n)],
                                 dref.at[pl.ds(doff, n)], gsem)
    for h in st.values():
        h.wait()
    plsc.subcore_barrier()

    def do_edges(src_h, dst_h, t0, t1, nchunks):
        def start_idx(i, b):
            off = (wid * nchunks + i) * CH
            return (
                pltpu.async_copy(src_h.at[pl.ds(off, CH)], src_v[b], isem),
                pltpu.async_copy(dst_h.at[pl.ds(off, CH)], dst_v[b], isem),
            )

        ih = {0: start_idx(0, 0)}
        sh = {}
        for i in range(nchunks):
            b = i % 2
            for h in ih.pop(i):
                h.wait()
            gh = (pltpu.async_copy(t0.at[src_v[b]], g0_v[b], gsem),
                  pltpu.async_copy(t1.at[src_v[b]], g1_v[b], gsem))
            if i - 1 in sh:
                for h in sh.pop(i - 1):
                    h.wait()
            if i + 1 < nchunks:
                ih[i + 1] = start_idx(i + 1, 1 - b)
            for h in gh:
                h.wait()
            sh[i] = (pltpu.async_copy(g0_v[b], acc0.at[dst_v[b]],
                                      ssem, add=True),
                     pltpu.async_copy(g1_v[b], acc1.at[dst_v[b]],
                                      ssem, add=True))
        for hs in sh.values():
            for h in hs:
                h.wait()

    do_edges(stj, dtj, ts0, ts1, TJ_CH)
    do_edges(sjj, djj, js0, js1, JJ_CH)
    plsc.subcore_barrier()
    h0 = pltpu.async_copy(acc0.at[pl.ds(base, RPT)],
                          stage_v.at[pl.ds(0, RPT)], gsem)
    h1 = pltpu.async_copy(acc1.at[pl.ds(base, RPT)],
                          g0_a.at[pl.ds(0, RPT)], gsem)
    h0.wait()
    s0 = pltpu.async_copy(stage_v.at[pl.ds(0, RPT)],
                          out0.at[pl.ds(hb, RPT)], ssem)
    h1.wait()
    s1 = pltpu.async_copy(g0_a.at[pl.ds(0, RPT)],
                          out1.at[pl.ds(hb, RPT)], ssem)
    s0.wait()
    s1.wait()


_sc_scatter = functools.partial(
    pl.kernel,
    mesh=plsc.VectorSubcoreMesh(core_axis_name="c", subcore_axis_name="s"),
    out_type=[jax.ShapeDtypeStruct((NC * NJP,), jnp.float32),
              jax.ShapeDtypeStruct((NC * NJP,), jnp.float32)],
    scratch_types=[
        pltpu.VMEM((CH,), jnp.int32),
        pltpu.VMEM((CH,), jnp.int32),
        pltpu.VMEM((CH,), jnp.int32),
        pltpu.VMEM((CH,), jnp.int32),
        pltpu.VMEM((CH,), jnp.float32),
        pltpu.VMEM((CH,), jnp.float32),
        pltpu.VMEM((CH,), jnp.float32),
        pltpu.VMEM((CH,), jnp.float32),
        pltpu.VMEM((RPT,), jnp.float32),
        pltpu.VMEM_SHARED((NJP,), jnp.float32),
        pltpu.VMEM_SHARED((NJP,), jnp.float32),
        pltpu.VMEM_SHARED((NTP,), jnp.float32),
        pltpu.VMEM_SHARED((NTP,), jnp.float32),
        pltpu.VMEM_SHARED((NJP,), jnp.float32),
        pltpu.VMEM_SHARED((NJP,), jnp.float32),
        pltpu.SemaphoreType.DMA,
        pltpu.SemaphoreType.DMA,
        pltpu.SemaphoreType.DMA,
    ],
)(_sc_body)


def kernel(x_joint, x_torso, edge_index_tj, edge_index_jt, edge_index_jj,
           Wj, bj, Wt, bt, Wrel_tj, brel_tj, Wroot_tj,
           Wrel_jt, brel_jt, Wroot_jt, Wrel_jj, brel_jj, Wroot_jj,
           Wout, bout):
    f32 = jnp.float32
    xjT = x_joint.T
    xtT = x_torso.T
    woutT = Wout.T
    brel_col = (brel_tj + brel_jj).reshape(64, 1)
    bj_col = bj.reshape(11, 1)
    bt_col = bt.reshape(11, 1)
    bout_col = bout.reshape(2, 1)

    blkj = 16000
    prej = pl.pallas_call(
        _pre_joint_body,
        grid=(N_J // blkj,),
        in_specs=[pl.BlockSpec((2, 64), lambda i: (0, 0)),
                  pl.BlockSpec((64, 11), lambda i: (0, 0)),
                  pl.BlockSpec((64, 11), lambda i: (0, 0)),
                  pl.BlockSpec((64, 11), lambda i: (0, 0)),
                  pl.BlockSpec((11, 2), lambda i: (0, 0)),
                  pl.BlockSpec((11, 1), lambda i: (0, 0)),
                  pl.BlockSpec((64, 1), lambda i: (0, 0)),
                  pl.BlockSpec((2, 1), lambda i: (0, 0)),
                  pl.BlockSpec((2, blkj), lambda i: (0, i))],
        out_specs=pl.BlockSpec((4, blkj), lambda i: (0, i)),
        out_shape=jax.ShapeDtypeStruct((4, N_J), f32),
    )(woutT, Wrel_jj.T, Wroot_tj.T, Wroot_jj.T, Wj.T,
      bj_col, brel_col, bout_col, xjT)

    pret = pl.pallas_call(
        _pre_torso_body,
        grid=(1,),
        in_specs=[pl.BlockSpec((2, 64), lambda i: (0, 0)),
                  pl.BlockSpec((64, 11), lambda i: (0, 0)),
                  pl.BlockSpec((11, 11), lambda i: (0, 0)),
                  pl.BlockSpec((11, 1), lambda i: (0, 0)),
                  pl.BlockSpec((11, N_T), lambda i: (0, 0))],
        out_specs=pl.BlockSpec((2, N_T), lambda i: (0, 0)),
        out_shape=jax.ShapeDtypeStruct((2, N_T), f32),
    )(woutT, Wrel_tj.T, Wt.T, bt_col, xtT)

    zpadj = jnp.zeros((_PAD,), f32)
    zpadt = jnp.zeros((_PADT,), f32)
    zrow = jnp.zeros((NJP,), f32)
    yj0 = jnp.concatenate([prej[0], zpadj])
    yj1 = jnp.concatenate([prej[1], zpadj])
    init0 = jnp.concatenate([prej[2], zpadj, zrow])
    init1 = jnp.concatenate([prej[3], zpadj, zrow])
    yt0 = jnp.concatenate([pret[0], zpadt])
    yt1 = jnp.concatenate([pret[1], zpadt])
    ar_tj = jnp.arange(E_TJ_P - edge_index_tj.shape[1], dtype=jnp.int32)
    ar_jj = jnp.arange(E_JJ_P - edge_index_jj.shape[1], dtype=jnp.int32)
    stj = jnp.concatenate([edge_index_tj[0].astype(jnp.int32),
                           N_T + ar_tj % _PADT])
    dtj = jnp.concatenate([edge_index_tj[1].astype(jnp.int32),
                           N_J + ar_tj % _PAD])
    sjj = jnp.concatenate([edge_index_jj[0].astype(jnp.int32),
                           N_J + ar_jj % _PAD])
    djj = jnp.concatenate([edge_index_jj[1].astype(jnp.int32),
                           N_J + ar_jj % _PAD])

    p0, p1 = _sc_scatter(yt0, yt1, yj0, yj1, stj, dtj, sjj, djj, init0, init1)
    p0 = p0.reshape(NC, NJP)
    p1 = p1.reshape(NC, NJP)

    blk = 16000
    loc2, scale2 = pl.pallas_call(
        _post_body,
        grid=(N_J // blk,),
        in_specs=[pl.BlockSpec((2, blk), lambda i: (0, i)),
                  pl.BlockSpec((2, blk), lambda i: (0, i))],
        out_specs=[pl.BlockSpec((1, blk), lambda i: (0, i)),
                   pl.BlockSpec((1, blk), lambda i: (0, i))],
        out_shape=[jax.ShapeDtypeStruct((1, N_J), f32),
                   jax.ShapeDtypeStruct((1, N_J), f32)],
    )(p0, p1)
    return (loc2.reshape(N_J), scale2.reshape(N_J))


# speedup vs baseline: 1.3085x; 1.0413x over previous
import functools

import numpy as np
import jax
import jax.numpy as jnp
from jax import lax
from jax.experimental import pallas as pl
from jax.experimental.pallas import tpu as pltpu
from jax.experimental.pallas import tpu_sc as plsc

N_J = 80000
N_T = 20000
_PAD = 2048
_PADT = 2528
NJP = N_J + _PAD
NTP = N_T + _PADT
NC = 2
NS = 16
NW = NC * NS
CH = 10240
TJ_CH = 2
JJ_CH = 6
E_TJ_P = NW * TJ_CH * CH
E_JJ_P = NW * JJ_CH * CH
RPT = NJP // NS
TPT = NTP // NS
_SP_BIAS = float(np.log(np.exp(1.0) - 1.0))

def _pre_joint_body(woutT_ref, wreljjT_ref, wrtjT_ref, wrjjT_ref, wjT_ref,
                    bj_ref, brel_ref, bout_ref, x_ref, o_ref, p_ref):
    ajjT = jnp.dot(woutT_ref[...], wreljjT_ref[...],
                   preferred_element_type=jnp.float32)
    arT = jnp.dot(woutT_ref[...], wrtjT_ref[...] + wrjjT_ref[...],
                  preferred_element_type=jnp.float32)
    gjT = jnp.dot(ajjT, wjT_ref[...], preferred_element_type=jnp.float32)
    grT = jnp.dot(arT, wjT_ref[...], preferred_element_type=jnp.float32)
    cj = jnp.dot(ajjT, bj_ref[...], preferred_element_type=jnp.float32)
    cr = (jnp.dot(arT, bj_ref[...], preferred_element_type=jnp.float32)
          + jnp.dot(woutT_ref[...], brel_ref[...],
                    preferred_element_type=jnp.float32)
          + bout_ref[...])
    g4 = jnp.concatenate([gjT, grT], axis=0)
    c4 = jnp.concatenate([cj, cr], axis=0)
    x = x_ref[...]
    o = jnp.dot(g4, x, preferred_element_type=jnp.float32) + c4
    o_ref[...] = o[2:4, :]
    u0 = lax.bitcast_convert_type(o[0:1, :].astype(jnp.bfloat16),
                                  jnp.uint16).astype(jnp.int32)
    u1 = lax.bitcast_convert_type(o[1:2, :].astype(jnp.bfloat16),
                                  jnp.uint16).astype(jnp.int32)
    p_ref[...] = u0 | (u1 << 16)


def _pre_torso_body(woutT_ref, wreltjT_ref, wtT_ref, bt_ref, x_ref, p_ref):
    atjT = jnp.dot(woutT_ref[...], wreltjT_ref[...],
                   preferred_element_type=jnp.float32)
    gtT = jnp.dot(atjT, wtT_ref[...], preferred_element_type=jnp.float32)
    ct = jnp.dot(atjT, bt_ref[...], preferred_element_type=jnp.float32)
    x = x_ref[...]
    o = jnp.dot(gtT, x, preferred_element_type=jnp.float32) + ct
    u0 = lax.bitcast_convert_type(o[0:1, :].astype(jnp.bfloat16),
                                  jnp.uint16).astype(jnp.int32)
    u1 = lax.bitcast_convert_type(o[1:2, :].astype(jnp.bfloat16),
                                  jnp.uint16).astype(jnp.int32)
    p_ref[...] = u0 | (u1 << 16)


def _post_body(p0_ref, p1_ref, loc_ref, scale_ref):
    loc_ref[...] = p0_ref[0:1, :] + p0_ref[1:2, :]
    s = p1_ref[0:1, :] + p1_ref[1:2, :] + _SP_BIAS
    scale_ref[...] = jax.nn.softplus(s)


def _sc_body(tp, jp, stj, dtj, sjj, djj, init0, init1,
             out0, out1,
             src_a, src_b, dst_a, dst_b, gp_a, gp_b, g0_a, g0_b, g1_a, g1_b,
             stage_v, acc0, acc1, tsp, jsp, isem, gsem, ssem):
    src_v = (src_a, src_b)
    dst_v = (dst_a, dst_b)
    gp_v = (gp_a, gp_b)
    g0_v = (g0_a, g0_b)
    g1_v = (g1_a, g1_b)
    c = lax.axis_index("c")
    s = lax.axis_index("s")
    wid = s * NC + c
    base = s * RPT
    tb = s * TPT
    hb = c * NJP + base

    jobs = ((init0, hb, stage_v, acc0, base, RPT),
            (init1, hb, g0_a, acc1, base, RPT),
            (jp, base, gp_a, jsp, base, RPT),
            (tp, tb, gp_b, tsp, tb, TPT))
    lds = [pltpu.async_copy(sref.at[pl.ds(so, n)], buf.at[pl.ds(0, n)], isem)
           for sref, so, buf, _, _, n in jobs]
    sts = []
    for k, (_, _, buf, dref, doff, n) in enumerate(jobs):
        lds[k].wait()
        sts.append(pltpu.async_copy(buf.at[pl.ds(0, n)],
                                    dref.at[pl.ds(doff, n)], gsem))
    for h in sts:
        h.wait()
    plsc.subcore_barrier()

    def do_edges(src_h, dst_h, tpk, nchunks):
        def start_idx(i, b):
            off = (wid * nchunks + i) * CH
            return (
                pltpu.async_copy(src_h.at[pl.ds(off, CH)], src_v[b], isem),
                pltpu.async_copy(dst_h.at[pl.ds(off, CH)], dst_v[b], isem),
            )

        ih = {0: start_idx(0, 0)}
        sh = {}
        for i in range(nchunks):
            b = i % 2
            for h in ih.pop(i):
                h.wait()
            gh = pltpu.async_copy(tpk.at[src_v[b]], gp_v[b], gsem)
            if i - 1 in sh:
                for h in sh.pop(i - 1):
                    h.wait()
            if i + 1 < nchunks:
                ih[i + 1] = start_idx(i + 1, 1 - b)
            gh.wait()

            gpb, g0b, g1b = gp_v[b], g0_v[b], g1_v[b]

            def unpack(k, carry):
                w = gpb[pl.ds(k * 16, 16)]
                g0b[pl.ds(k * 16, 16)] = lax.bitcast_convert_type(
                    w << 16, jnp.float32)
                g1b[pl.ds(k * 16, 16)] = lax.bitcast_convert_type(
                    w & jnp.int32(-65536), jnp.float32)
                return carry

            lax.fori_loop(0, CH // 16, unpack, 0, unroll=8)
            sh[i] = (pltpu.async_copy(g0_v[b], acc0.at[dst_v[b]],
                                      ssem, add=True),
                     pltpu.async_copy(g1_v[b], acc1.at[dst_v[b]],
                                      ssem, add=True))
        for hs in sh.values():
            for h in hs:
                h.wait()

    do_edges(stj, dtj, tsp, TJ_CH)
    do_edges(sjj, djj, jsp, JJ_CH)
    plsc.subcore_barrier()
    h0 = pltpu.async_copy(acc0.at[pl.ds(base, RPT)],
                          stage_v.at[pl.ds(0, RPT)], gsem)
    h1 = pltpu.async_copy(acc1.at[pl.ds(base, RPT)],
                          g0_a.at[pl.ds(0, RPT)], gsem)
    h0.wait()
    s0 = pltpu.async_copy(stage_v.at[pl.ds(0, RPT)],
                          out0.at[pl.ds(hb, RPT)], ssem)
    h1.wait()
    s1 = pltpu.async_copy(g0_a.at[pl.ds(0, RPT)],
                          out1.at[pl.ds(hb, RPT)], ssem)
    s0.wait()
    s1.wait()


_sc_scatter = functools.partial(
    pl.kernel,
    mesh=plsc.VectorSubcoreMesh(core_axis_name="c", subcore_axis_name="s"),
    out_type=[jax.ShapeDtypeStruct((NC * NJP,), jnp.float32),
              jax.ShapeDtypeStruct((NC * NJP,), jnp.float32)],
    scratch_types=[
        pltpu.VMEM((CH,), jnp.int32),
        pltpu.VMEM((CH,), jnp.int32),
        pltpu.VMEM((CH,), jnp.int32),
        pltpu.VMEM((CH,), jnp.int32),
        pltpu.VMEM((CH,), jnp.int32),
        pltpu.VMEM((CH,), jnp.int32),
        pltpu.VMEM((CH,), jnp.float32),
        pltpu.VMEM((CH,), jnp.float32),
        pltpu.VMEM((CH,), jnp.float32),
        pltpu.VMEM((CH,), jnp.float32),
        pltpu.VMEM((RPT,), jnp.float32),
        pltpu.VMEM_SHARED((NJP,), jnp.float32),
        pltpu.VMEM_SHARED((NJP,), jnp.float32),
        pltpu.VMEM_SHARED((NTP,), jnp.int32),
        pltpu.VMEM_SHARED((NJP,), jnp.int32),
        pltpu.SemaphoreType.DMA,
        pltpu.SemaphoreType.DMA,
        pltpu.SemaphoreType.DMA,
    ],
)(_sc_body)


def kernel(x_joint, x_torso, edge_index_tj, edge_index_jt, edge_index_jj,
           Wj, bj, Wt, bt, Wrel_tj, brel_tj, Wroot_tj,
           Wrel_jt, brel_jt, Wroot_jt, Wrel_jj, brel_jj, Wroot_jj,
           Wout, bout):
    f32 = jnp.float32
    xjT = x_joint.T
    xtT = x_torso.T
    woutT = Wout.T
    brel_col = (brel_tj + brel_jj).reshape(64, 1)
    bj_col = bj.reshape(11, 1)
    bt_col = bt.reshape(11, 1)
    bout_col = bout.reshape(2, 1)

    blkj = 16000
    prej = pl.pallas_call(
        _pre_joint_body,
        grid=(N_J // blkj,),
        in_specs=[pl.BlockSpec((2, 64), lambda i: (0, 0)),
                  pl.BlockSpec((64, 11), lambda i: (0, 0)),
                  pl.BlockSpec((64, 11), lambda i: (0, 0)),
                  pl.BlockSpec((64, 11), lambda i: (0, 0)),
                  pl.BlockSpec((11, 2), lambda i: (0, 0)),
                  pl.BlockSpec((11, 1), lambda i: (0, 0)),
                  pl.BlockSpec((64, 1), lambda i: (0, 0)),
                  pl.BlockSpec((2, 1), lambda i: (0, 0)),
                  pl.BlockSpec((2, blkj), lambda i: (0, i))],
        out_specs=[pl.BlockSpec((2, blkj), lambda i: (0, i)),
                   pl.BlockSpec((1, blkj), lambda i: (0, i))],
        out_shape=[jax.ShapeDtypeStruct((2, N_J), f32),
                   jax.ShapeDtypeStruct((1, N_J), jnp.int32)],
    )(woutT, Wrel_jj.T, Wroot_tj.T, Wroot_jj.T, Wj.T,
      bj_col, brel_col, bout_col, xjT)

    pret = pl.pallas_call(
        _pre_torso_body,
        grid=(1,),
        in_specs=[pl.BlockSpec((2, 64), lambda i: (0, 0)),
                  pl.BlockSpec((64, 11), lambda i: (0, 0)),
                  pl.BlockSpec((11, 11), lambda i: (0, 0)),
                  pl.BlockSpec((11, 1), lambda i: (0, 0)),
                  pl.BlockSpec((11, N_T), lambda i: (0, 0))],
        out_specs=pl.BlockSpec((1, N_T), lambda i: (0, 0)),
        out_shape=jax.ShapeDtypeStruct((1, N_T), jnp.int32),
    )(woutT, Wrel_tj.T, Wt.T, bt_col, xtT)

    roots, jpacked = prej
    zpadj = jnp.zeros((_PAD,), f32)
    zrow = jnp.zeros((NJP,), f32)
    jp = jnp.concatenate([jpacked[0], jnp.zeros((_PAD,), jnp.int32)])
    tp = jnp.concatenate([pret[0], jnp.zeros((_PADT,), jnp.int32)])
    init0 = jnp.concatenate([roots[0], zpadj, zrow])
    init1 = jnp.concatenate([roots[1], zpadj, zrow])
    ar_tj = jnp.arange(E_TJ_P - edge_index_tj.shape[1], dtype=jnp.int32)
    ar_jj = jnp.arange(E_JJ_P - edge_index_jj.shape[1], dtype=jnp.int32)
    stj = jnp.concatenate([edge_index_tj[0].astype(jnp.int32),
                           N_T + ar_tj % _PADT])
    dtj = jnp.concatenate([edge_index_tj[1].astype(jnp.int32),
                           N_J + ar_tj % _PAD])
    sjj = jnp.concatenate([edge_index_jj[0].astype(jnp.int32),
                           N_J + ar_jj % _PAD])
    djj = jnp.concatenate([edge_index_jj[1].astype(jnp.int32),
                           N_J + ar_jj % _PAD])

    p0, p1 = _sc_scatter(tp, jp, stj, dtj, sjj, djj, init0, init1)
    p0 = p0.reshape(NC, NJP)
    p1 = p1.reshape(NC, NJP)

    blk = 16000
    loc2, scale2 = pl.pallas_call(
        _post_body,
        grid=(N_J // blk,),
        in_specs=[pl.BlockSpec((2, blk), lambda i: (0, i)),
                  pl.BlockSpec((2, blk), lambda i: (0, i))],
        out_specs=[pl.BlockSpec((1, blk), lambda i: (0, i)),
                   pl.BlockSpec((1, blk), lambda i: (0, i))],
        out_shape=[jax.ShapeDtypeStruct((1, N_J), f32),
                   jax.ShapeDtypeStruct((1, N_J), f32)],
    )(p0, p1)
    return (loc2.reshape(N_J), scale2.reshape(N_J))
